# baseline (speedup 1.0000x reference)
import jax
import jax.numpy as jnp
from jax import lax
from jax.experimental import pallas as pl
from jax.experimental.pallas import tpu as pltpu
from jax.experimental.pallas import tpu_sc as plsc

_B = 4096
_L = 20
_D = 64
_LANES = 16
_NC = 2
_NS = 16
_NW = _NC * _NS
_BPW = _B // _NW
_G = _BPW // _LANES
_L2 = 1e-05

_N = 100000
_HB = 256
_NBLK = 196
_NL = _NBLK * _HB


def _repack_body(ua_ref, ub_ref, ba_ref, bb_ref, uo_ref, bo_ref):
    uo_ref[...] = jnp.concatenate(
        [jnp.transpose(ua_ref[...], (1, 0)),
         jnp.transpose(ub_ref[...], (1, 0))], axis=1)
    bo_ref[...] = jnp.concatenate(
        [jnp.transpose(ba_ref[...], (1, 0)),
         jnp.transpose(bb_ref[...], (1, 0))], axis=1)


def _repack(uf_t, bf_t):
    in_spec_a = pl.BlockSpec((_D, _HB), lambda i: (0, 2 * i))
    in_spec_b = pl.BlockSpec((_D, _HB), lambda i: (0, 2 * i + 1))
    return pl.pallas_call(
        _repack_body,
        grid=(_NBLK,),
        in_specs=[in_spec_a, in_spec_b, in_spec_a, in_spec_b],
        out_specs=[
            pl.BlockSpec((_HB, 2 * _D), lambda i: (i, 0)),
            pl.BlockSpec((_HB, 2 * _D), lambda i: (i, 0)),
        ],
        out_shape=[
            jax.ShapeDtypeStruct((_NL, 2 * _D), jnp.float32),
            jax.ShapeDtypeStruct((_NL, 2 * _D), jnp.float32),
        ],
    )(uf_t, uf_t, bf_t, bf_t)


def _line_of(u):
    line = ((u >> 9) << 8) + (u & 255)
    hoff = ((u >> 8) & 1) * _D
    return line, hoff


def _sc_body(users_hbm, bundles_hbm, ufp_hbm, bfp_hbm,
             pred_hbm, part_hbm,
             uidx_v, uoff_v, bidx_v, idxt_v, offt_v,
             urows_v, brows_v, pbuf_v, pvec_v, usem, bsem):
    cid = lax.axis_index("c")
    sid = lax.axis_index("s")
    wid = sid * _NC + cid
    base = wid * _BPW
    iota = lax.iota(jnp.int32, _LANES)

    pltpu.sync_copy(users_hbm.at[pl.ds(base, _BPW)], uidx_v)
    pltpu.sync_copy(bundles_hbm.at[pl.ds(base * _L, _BPW * _L)], bidx_v)

    for g in range(_G):
        u = uidx_v[pl.ds(g * _LANES, _LANES)]
        line, hoff = _line_of(u)
        uidx_v[pl.ds(g * _LANES, _LANES)] = line
        uoff_v[pl.ds(g * _LANES, _LANES)] = hoff

    udma = pltpu.async_copy(ufp_hbm.at[uidx_v], urows_v, usem)

    def tbody(l, carry):
        ls = jnp.full((_LANES,), l, jnp.int32)
        for g in range(_G):
            src = (iota + (g * _LANES)) * _L + ls
            u = plsc.load_gather(bidx_v, [src])
            line, hoff = _line_of(u)
            dst = pl.ds(l * _BPW + g * _LANES, _LANES)
            idxt_v[dst] = line
            offt_v[dst] = hoff
        return carry
    lax.fori_loop(0, _L, tbody, jnp.int32(0))

    pltpu.async_copy(
        bfp_hbm.at[idxt_v.at[pl.ds(0, _BPW)]],
        brows_v.at[pl.ds(0, _BPW)], bsem.at[0])

    udma.wait()

    def ubody(g, usq):
        rows = iota + (g * _LANES)
        hv = uoff_v[pl.ds(g * _LANES, _LANES)]
        for j in range(_D):
            cols = iota + j
            if j + _LANES > _D:
                cols = lax.bitwise_and(cols, _D - 1)
            v = plsc.load_gather(urows_v, [rows, hv + cols])
            usq = usq + v * v
        return usq
    usq = lax.fori_loop(0, _G, ubody, jnp.zeros((_LANES,), jnp.float32))

    def body(l, bsq):
        cur = lax.rem(l, 2)
        nxt = 1 - cur

        @pl.when(l + 1 < _L)
        def _prefetch():
            pltpu.async_copy(
                bfp_hbm.at[idxt_v.at[pl.ds((l + 1) * _BPW, _BPW)]],
                brows_v.at[pl.ds(nxt * _BPW, _BPW)], bsem.at[nxt])

        pltpu.make_async_copy(
            bfp_hbm.at[idxt_v.at[pl.ds(l * _BPW, _BPW)]],
            brows_v.at[pl.ds(cur * _BPW, _BPW)], bsem.at[cur]).wait()

        ls = jnp.full((_LANES,), l, jnp.int32)
        roff = cur * _BPW
        for g in range(_G):
            rows = iota + (g * _LANES)
            brows = rows + roff
            uhv = uoff_v[pl.ds(g * _LANES, _LANES)]
            bhv = offt_v[pl.ds(l * _BPW + g * _LANES, _LANES)]
            acc = jnp.zeros((_LANES,), jnp.float32)
            for j in range(_D):
                cols = iota + j
                if j + _LANES > _D:
                    cols = lax.bitwise_and(cols, _D - 1)
                uv = plsc.load_gather(urows_v, [rows, uhv + cols])
                bv = plsc.load_gather(brows_v, [brows, bhv + cols])
                acc = acc + uv * bv
                bsq = bsq + bv * bv
            plsc.store_scatter(pbuf_v, [rows * _L + ls], acc)
        return bsq
    bsq = lax.fori_loop(0, _L, body, jnp.zeros((_LANES,), jnp.float32))

    pltpu.sync_copy(pbuf_v, pred_hbm.at[pl.ds(base * _L, _BPW * _L)])

    pvec_v[...] = jnp.float32(_L) * usq + bsq
    pltpu.sync_copy(pvec_v, part_hbm.at[pl.ds(wid * _LANES, _LANES)])


_sc_kernel = pl.kernel(
    _sc_body,
    out_type=[
        jax.ShapeDtypeStruct((_B * _L,), jnp.float32),
        jax.ShapeDtypeStruct((_NW * _LANES,), jnp.float32),
    ],
    mesh=plsc.VectorSubcoreMesh(core_axis_name="c", subcore_axis_name="s"),
    compiler_params=pltpu.CompilerParams(
        needs_layout_passes=False, use_tc_tiling_on_sc=False),
    scratch_types=[
        pltpu.VMEM((_BPW,), jnp.int32),
        pltpu.VMEM((_BPW,), jnp.int32),
        pltpu.VMEM((_BPW * _L,), jnp.int32),
        pltpu.VMEM((_L * _BPW,), jnp.int32),
        pltpu.VMEM((_L * _BPW,), jnp.int32),
        pltpu.VMEM((_BPW, 2 * _D), jnp.float32),
        pltpu.VMEM((2 * _BPW, 2 * _D), jnp.float32),
        pltpu.VMEM((_BPW * _L,), jnp.float32),
        pltpu.VMEM((_LANES,), jnp.float32),
        pltpu.SemaphoreType.DMA,
        pltpu.SemaphoreType.DMA((2,)),
    ],
)


@jax.jit
def kernel(users, bundles, users_feature, bundles_feature):
    ufp, bfp = _repack(users_feature.T, bundles_feature.T)
    pred_flat, parts = _sc_kernel(
        users.reshape(_B), bundles.reshape(_B * _L), ufp, bfp)
    pred = pred_flat.reshape(_B, _L)
    loss = jnp.float32(_L2) * jnp.sum(parts)
    return (pred, loss)
